# Initial kernel scaffold; baseline (speedup 1.0000x reference)
import jax
import jax.numpy as jnp
from jax.experimental import pallas as pl


def kernel(x, tau):
    raise NotImplementedError("write your pallas kernel here")


# speedup vs baseline: 13.6271x; 13.6271x over previous
import functools

import jax
import jax.numpy as jnp
from jax.experimental import pallas as pl
from jax.experimental.pallas import tpu as pltpu


def _tc_body(scale_ref, x_ref, o_ref, *, k: int):
    xb = x_ref[...]
    a = jax.lax.bitcast_convert_type(jnp.abs(xb), jnp.int32)
    r = xb.shape[0]
    t = jnp.zeros((r, 1), jnp.int32)
    for b in range(30, -1, -1):
        trial = t + (1 << b)
        cnt = jnp.sum((a >= trial).astype(jnp.int32), axis=1, keepdims=True)
        t = jnp.where(cnt >= k, trial, t)
    sparse = jnp.where(a >= t, xb, jnp.zeros_like(xb))
    o_ref[...] = sparse * scale_ref[0] + xb * scale_ref[1]


def kernel(x, tau):
    n, c, h, w = x.shape
    hw = h * w
    k = max(int(0.1 * hw), 1)
    rows = n * c
    x2 = x.reshape(rows, hw)

    blk = 256
    while rows % blk:
        blk //= 2
    grid = rows // blk

    tau_f = jnp.asarray(tau, x.dtype)
    is_id = tau_f == jnp.asarray(1.0, x.dtype)
    alpha = jnp.where(is_id, jnp.asarray(1.0, x.dtype), tau_f)
    beta = jnp.where(is_id, jnp.asarray(0.0, x.dtype), 1.0 - tau_f)
    scale = jnp.stack([alpha, beta])

    out = pl.pallas_call(
        functools.partial(_tc_body, k=k),
        grid=(grid,),
        in_specs=[
            pl.BlockSpec(memory_space=pltpu.SMEM),
            pl.BlockSpec((blk, hw), lambda i: (i, 0)),
        ],
        out_specs=pl.BlockSpec((blk, hw), lambda i: (i, 0)),
        out_shape=jax.ShapeDtypeStruct((rows, hw), x.dtype),
    )(scale, x2)
    return out.reshape(n, c, h, w)
